# baseline (speedup 1.0000x reference)
import functools
import math

import jax
import jax.numpy as jnp
from jax import lax
from jax.experimental import pallas as pl
from jax.experimental.pallas import tpu as pltpu
from jax.experimental.pallas import tpu_sc as plsc

D_MODEL = 64
SCALE = math.sqrt(D_MODEL)

_info = plsc.get_sparse_core_info()
_NC = _info.num_cores
_NS = _info.num_subcores
_NW = _NC * _NS
_L = _info.num_lanes

_CH = 512
_GSZ = 128
_NG = _CH // _GSZ


@functools.lru_cache(maxsize=None)
def _build(B: int):
    assert B % (_NW * _CH) == 0, B
    b_per_w = B // _NW
    n_chunks = b_per_w // _CH

    mesh = plsc.VectorSubcoreMesh(core_axis_name="c", subcore_axis_name="s")

    @functools.partial(
        pl.kernel,
        mesh=mesh,
        out_type=jax.ShapeDtypeStruct((B, D_MODEL), jnp.float32),
        scratch_types=[
            pltpu.VMEM((_CH,), jnp.int32),
            pltpu.VMEM((_CH, D_MODEL), jnp.float32),
            pltpu.SemaphoreType.DMA,
        ],
        compiler_params=pltpu.CompilerParams(use_tc_tiling_on_sc=False),
    )
    def emb(x_hbm, table_hbm, out_hbm, idx_v, rows_v, sem):
        wid = lax.axis_index("s") * _NC + lax.axis_index("c")
        base = wid * b_per_w

        def chunk(g, carry):
            row0 = base + g * _CH
            pltpu.sync_copy(x_hbm.at[pl.ds(row0, _CH)], idx_v)
            for j in range(_NG):
                pltpu.async_copy(
                    table_hbm.at[idx_v.at[pl.ds(j * _GSZ, _GSZ)]],
                    rows_v.at[pl.ds(j * _GSZ, _GSZ)],
                    sem,
                )
            for j in range(_NG):
                pltpu.make_async_copy(
                    table_hbm.at[idx_v.at[pl.ds(j * _GSZ, _GSZ)]],
                    rows_v.at[pl.ds(j * _GSZ, _GSZ)],
                    sem,
                ).wait()

            def scale_row(r, c2):
                for c in range(D_MODEL // _L):
                    sl = pl.ds(c * _L, _L)
                    rows_v[r, sl] = rows_v[r, sl] * SCALE
                return c2

            lax.fori_loop(0, _CH, scale_row, 0)
            pltpu.sync_copy(rows_v, out_hbm.at[pl.ds(row0, _CH)])
            return carry

        lax.fori_loop(0, n_chunks, chunk, 0)

    return emb


def kernel(x, table):
    B0, B1 = x.shape
    flat = x.reshape(B0 * B1).astype(jnp.int32)
    out = _build(B0 * B1)(flat, table)
    return out.reshape(B0, B1, D_MODEL)


# speedup vs baseline: 1.1393x; 1.1393x over previous
import functools
import math

import jax
import jax.numpy as jnp
from jax import lax
from jax.experimental import pallas as pl
from jax.experimental.pallas import tpu as pltpu
from jax.experimental.pallas import tpu_sc as plsc

D_MODEL = 64
SCALE = math.sqrt(D_MODEL)

_info = plsc.get_sparse_core_info()
_NC = _info.num_cores
_NS = _info.num_subcores
_NW = _NC * _NS
_L = _info.num_lanes

_CH = 256
_GSZ = 128
_NG = _CH // _GSZ
_AD = 4
_BD = 2


@functools.lru_cache(maxsize=None)
def _build(B: int):
    assert B % (_NW * _CH * _AD) == 0, B
    b_per_w = B // _NW
    n_chunks = b_per_w // _CH
    n_steps = n_chunks // _AD

    mesh = plsc.VectorSubcoreMesh(core_axis_name="c", subcore_axis_name="s")

    @functools.partial(
        pl.kernel,
        mesh=mesh,
        out_type=jax.ShapeDtypeStruct((B, D_MODEL), jnp.float32),
        scratch_types=[pltpu.VMEM((b_per_w,), jnp.int32)]
        + [pltpu.VMEM((_CH, D_MODEL), jnp.float32)] * (_AD + _BD)
        + [pltpu.SemaphoreType.DMA] * (_AD + _BD),
        compiler_params=pltpu.CompilerParams(use_tc_tiling_on_sc=False),
    )
    def emb(x_hbm, table_hbm, out_hbm, idx_v, a0, a1, a2, a3, b0, b1,
            sg0, sg1, sg2, sg3, ss0, ss1):
        abuf = [a0, a1, a2, a3]
        bbuf = [b0, b1]
        sg = [sg0, sg1, sg2, sg3]
        ss = [ss0, ss1]

        wid = lax.axis_index("s") * _NC + lax.axis_index("c")
        base = wid * b_per_w
        pltpu.sync_copy(x_hbm.at[pl.ds(base, b_per_w)], idx_v)

        def fire_gather(g, buf, sem):
            for t in range(_NG):
                pltpu.async_copy(
                    table_hbm.at[idx_v.at[pl.ds(g * _CH + t * _GSZ, _GSZ)]],
                    buf.at[pl.ds(t * _GSZ, _GSZ)],
                    sem,
                )

        def drain_gather(g, buf, sem):
            for t in range(_NG):
                pltpu.make_async_copy(
                    table_hbm.at[idx_v.at[pl.ds(g * _CH + t * _GSZ, _GSZ)]],
                    buf.at[pl.ds(t * _GSZ, _GSZ)],
                    sem,
                ).wait()

        for j in range(_AD):
            fire_gather(j, abuf[j], sg[j])

        @pl.loop(0, n_steps)
        def step(s):
            for j in range(_AD):
                g = s * _AD + j
                drain_gather(g, abuf[j], sg[j])

                def wait_scatter():
                    pltpu.make_async_copy(
                        bbuf[j % _BD],
                        out_hbm.at[pl.ds(base + (g - _BD) * _CH, _CH)],
                        ss[j % _BD],
                    ).wait()

                if j < _BD:
                    pl.when(s > 0)(wait_scatter)
                else:
                    wait_scatter()

                src = abuf[j]
                dst = bbuf[j % _BD]

                @plsc.parallel_loop(0, _CH, 1, unroll=8)
                def scale_row(r):
                    for c in range(D_MODEL // _L):
                        sl = pl.ds(c * _L, _L)
                        dst[r, sl] = src[r, sl] * SCALE

                pltpu.async_copy(
                    dst, out_hbm.at[pl.ds(base + g * _CH, _CH)], ss[j % _BD]
                )

                def refire(g=g, j=j):
                    fire_gather(g + _AD, abuf[j], sg[j])

                pl.when(s < n_steps - 1)(refire)

        for j in range(_BD):
            pltpu.make_async_copy(
                bbuf[j],
                out_hbm.at[pl.ds(base + (n_chunks - _BD + j) * _CH, _CH)],
                ss[j],
            ).wait()

    return emb


def kernel(x, table):
    B0, B1 = x.shape
    flat = x.reshape(B0 * B1).astype(jnp.int32)
    out = _build(B0 * B1)(flat, table)
    return out.reshape(B0, B1, D_MODEL)
